# baseline (speedup 1.0000x reference)
import functools

import jax
import jax.numpy as jnp
from jax import lax
from jax.experimental import pallas as pl
from jax.experimental.pallas import tpu as pltpu
from jax.experimental.pallas import tpu_sc as plsc

N_NODE = 10000
D = 128
E_EDGES = 320000

NS = 16
N_PAD = 10240
ROWS_PER_TILE = N_PAD // NS
CHUNK = 125
ROWS_PER_REL = E_EDGES // CHUNK
CHUNKS_PER_TILE = ROWS_PER_REL // NS
NI = 16
NBLK = CHUNKS_PER_TILE // NI
NITER = CHUNKS_PER_TILE // 2


@functools.partial(
    pl.kernel,
    mesh=plsc.VectorSubcoreMesh(core_axis_name="c", subcore_axis_name="s"),
    out_type=jax.ShapeDtypeStruct((2 * N_PAD, D), jnp.float32),
    scratch_types=[
        pltpu.VMEM((2, NI, CHUNK), jnp.int32),
        pltpu.VMEM((2, NI, CHUNK), jnp.int32),
        pltpu.VMEM((CHUNK, D), jnp.float32),
        pltpu.VMEM((CHUNK, D), jnp.float32),
        pltpu.VMEM_SHARED((N_PAD, D), jnp.float32),
        pltpu.SemaphoreType.DMA,
        pltpu.SemaphoreType.DMA,
        pltpu.SemaphoreType.DMA,
        pltpu.SemaphoreType.DMA,
        pltpu.SemaphoreType.DMA,
    ],
)
def _sc_agg(x_hbm, src_hbm, dst_hbm, zeros_hbm,
            agg_hbm, sidx, didx, r0, r1, aggsh, g0, g1, s0, s1, isem):
    c = lax.axis_index("c")
    s = lax.axis_index("s")

    rowbase = s * ROWS_PER_TILE
    pltpu.sync_copy(zeros_hbm, aggsh.at[pl.ds(rowbase, ROWS_PER_TILE)])
    plsc.subcore_barrier()

    tbase = c * ROWS_PER_REL + s * CHUNKS_PER_TILE

    def _load_block(b, slot, sem):
        pltpu.async_copy(src_hbm.at[pl.ds(tbase + NI * b, NI)], sidx.at[slot],
                         sem)
        pltpu.async_copy(dst_hbm.at[pl.ds(tbase + NI * b, NI)], didx.at[slot],
                         sem)

    def _wait_block(slot, sem):
        pltpu.make_async_copy(src_hbm.at[pl.ds(tbase, NI)], sidx.at[slot],
                              sem).wait()
        pltpu.make_async_copy(dst_hbm.at[pl.ds(tbase, NI)], didx.at[slot],
                              sem).wait()

    _load_block(0, 0, isem)
    _wait_block(0, isem)
    _load_block(1, 1, isem)

    pltpu.async_copy(x_hbm.at[sidx.at[0, 0]], r0, g0)
    pltpu.async_copy(x_hbm.at[sidx.at[0, 1]], r1, g1)

    def _steady(jj, carry):
        slot = (jj // 8) % 2
        row0 = 2 * (jj % 8)
        pltpu.make_async_copy(x_hbm.at[sidx.at[slot, row0]], r0, g0).wait()
        pltpu.async_copy(r0, aggsh.at[didx.at[slot, row0]], s0, add=True)
        pltpu.make_async_copy(x_hbm.at[sidx.at[slot, row0 + 1]], r1, g1).wait()
        pltpu.async_copy(r1, aggsh.at[didx.at[slot, row0 + 1]], s1, add=True)

        @pl.when(jj < NITER - 1)
        def _():
            pltpu.make_async_copy(r0, aggsh.at[didx.at[slot, row0]], s0).wait()
            pltpu.make_async_copy(r1, aggsh.at[didx.at[slot, row0 + 1]],
                                  s1).wait()

            @pl.when(jnp.logical_and(jj % 8 == 7, jj < 8 * (NBLK - 2)))
            def _():
                _load_block(jj // 8 + 2, slot, isem)

            @pl.when(jj % 8 == 7)
            def _():
                _wait_block(1 - slot, isem)

            nslot = ((jj + 1) // 8) % 2
            nrow0 = 2 * ((jj + 1) % 8)
            pltpu.async_copy(x_hbm.at[sidx.at[nslot, nrow0]], r0, g0)
            pltpu.async_copy(x_hbm.at[sidx.at[nslot, nrow0 + 1]], r1, g1)

        return carry

    lax.fori_loop(0, NITER, _steady, 0)
    pltpu.make_async_copy(r0, aggsh.at[didx.at[1, NI - 2]], s0).wait()
    pltpu.make_async_copy(r1, aggsh.at[didx.at[1, NI - 1]], s1).wait()
    plsc.subcore_barrier()

    outbase = c * N_PAD + rowbase
    pltpu.sync_copy(aggsh.at[pl.ds(rowbase, ROWS_PER_TILE)],
                    agg_hbm.at[pl.ds(outbase, ROWS_PER_TILE)])


BLK = 80
UOFF = N_PAD // BLK


def _dense_body(agg, x_u, x_i, wl_u, b_u, wr_u, wl_i, b_i, wr_i,
                out_u, out_i):
    f32 = jnp.float32
    out_u[...] = (jnp.dot(agg[0], wl_u[...], preferred_element_type=f32)
                  + jnp.dot(x_u[...], wr_u[...], preferred_element_type=f32)
                  + b_u[...])
    out_i[...] = (jnp.dot(agg[1], wl_i[...], preferred_element_type=f32)
                  + jnp.dot(x_i[...], wr_i[...], preferred_element_type=f32)
                  + b_i[...])


def _dense(agg_flat, x_user, x_item,
           WlT_u, b_u, WrT_u, WlT_i, b_i, WrT_i):
    agg3 = agg_flat.reshape(2, N_PAD, D)
    agg_spec = pl.BlockSpec((2, BLK, D), lambda i: (0, i, 0))
    row_spec = pl.BlockSpec((BLK, D), lambda i: (i, 0))
    w_spec = pl.BlockSpec((D, D), lambda i: (0, 0))
    b_spec = pl.BlockSpec((1, D), lambda i: (0, 0))
    return pl.pallas_call(
        _dense_body,
        grid=(N_NODE // BLK,),
        in_specs=[agg_spec, row_spec, row_spec,
                  w_spec, b_spec, w_spec, w_spec, b_spec, w_spec],
        out_specs=[row_spec, row_spec],
        out_shape=[jax.ShapeDtypeStruct((N_NODE, D), jnp.float32),
                   jax.ShapeDtypeStruct((N_NODE, D), jnp.float32)],
    )(agg3, x_user, x_item,
      WlT_u, b_u, WrT_u, WlT_i, b_i, WrT_i)


def kernel(x_user, x_item, edge_index_user_to_item, edge_index_item_to_user,
           W_l_u2i, b_l_u2i, W_r_u2i, W_l_i2u, b_l_i2u, W_r_i2u):
    x_all = jnp.concatenate([x_user, x_item], axis=0)
    src_all = jnp.concatenate([edge_index_user_to_item[0],
                               edge_index_item_to_user[0] + N_NODE]
                              ).reshape(2 * ROWS_PER_REL, CHUNK)
    dst_all = jnp.concatenate([edge_index_user_to_item[1],
                               edge_index_item_to_user[1]]
                              ).reshape(2 * ROWS_PER_REL, CHUNK)
    zeros_rows = jnp.zeros((ROWS_PER_TILE, D), jnp.float32)

    agg_flat = _sc_agg(x_all, src_all, dst_all, zeros_rows)

    out_user, out_item = _dense(
        agg_flat, x_user, x_item,
        W_l_i2u.T, b_l_i2u.reshape(1, D), W_r_i2u.T,
        W_l_u2i.T, b_l_u2i.reshape(1, D), W_r_u2i.T)
    return out_user, out_item


# speedup vs baseline: 1.0007x; 1.0007x over previous
import functools

import jax
import jax.numpy as jnp
from jax import lax
from jax.experimental import pallas as pl
from jax.experimental.pallas import tpu as pltpu
from jax.experimental.pallas import tpu_sc as plsc

N_NODE = 10000
D = 128
E_EDGES = 320000

NS = 16
N_PAD = 10240
ROWS_PER_TILE = N_PAD // NS
CHUNK = 125
ROWS_PER_REL = E_EDGES // CHUNK
CHUNKS_PER_TILE = ROWS_PER_REL // NS
NI = 16
NBLK = CHUNKS_PER_TILE // NI
NITER = CHUNKS_PER_TILE // 2


@functools.partial(
    pl.kernel,
    mesh=plsc.VectorSubcoreMesh(core_axis_name="c", subcore_axis_name="s"),
    out_type=jax.ShapeDtypeStruct((2 * N_PAD, D), jnp.float32),
    scratch_types=[
        pltpu.VMEM((2, NI, CHUNK), jnp.int32),
        pltpu.VMEM((2, NI, CHUNK), jnp.int32),
        pltpu.VMEM((CHUNK, D), jnp.float32),
        pltpu.VMEM((CHUNK, D), jnp.float32),
        pltpu.VMEM_SHARED((N_PAD, D), jnp.float32),
        pltpu.SemaphoreType.DMA,
        pltpu.SemaphoreType.DMA,
        pltpu.SemaphoreType.DMA,
        pltpu.SemaphoreType.DMA,
        pltpu.SemaphoreType.DMA,
    ],
)
def _sc_agg(x_hbm, src_hbm, dst_hbm, zeros_hbm,
            agg_hbm, sidx, didx, r0, r1, aggsh, g0, g1, s0, s1, isem):
    c = lax.axis_index("c")
    s = lax.axis_index("s")

    rowbase = s * ROWS_PER_TILE
    pltpu.sync_copy(zeros_hbm, aggsh.at[pl.ds(rowbase, ROWS_PER_TILE)])
    plsc.subcore_barrier()

    tbase = c * ROWS_PER_REL + s * CHUNKS_PER_TILE

    def _load_block(b, slot, sem):
        pltpu.async_copy(src_hbm.at[pl.ds(tbase + NI * b, NI)], sidx.at[slot],
                         sem)
        pltpu.async_copy(dst_hbm.at[pl.ds(tbase + NI * b, NI)], didx.at[slot],
                         sem)

    def _wait_block(slot, sem):
        pltpu.make_async_copy(src_hbm.at[pl.ds(tbase, NI)], sidx.at[slot],
                              sem).wait()
        pltpu.make_async_copy(dst_hbm.at[pl.ds(tbase, NI)], didx.at[slot],
                              sem).wait()

    _load_block(0, 0, isem)
    _wait_block(0, isem)
    _load_block(1, 1, isem)

    pltpu.async_copy(x_hbm.at[sidx.at[0, 0]], r0, g0)
    pltpu.async_copy(x_hbm.at[sidx.at[0, 1]], r1, g1)

    def _steady(jj, carry):
        slot = (jj // 8) % 2
        row0 = 2 * (jj % 8)
        pltpu.make_async_copy(x_hbm.at[sidx.at[slot, row0]], r0, g0).wait()
        pltpu.async_copy(r0, aggsh.at[didx.at[slot, row0]], s0, add=True)
        pltpu.make_async_copy(x_hbm.at[sidx.at[slot, row0 + 1]], r1, g1).wait()
        pltpu.async_copy(r1, aggsh.at[didx.at[slot, row0 + 1]], s1, add=True)

        @pl.when(jj < NITER - 1)
        def _():
            pltpu.make_async_copy(r0, aggsh.at[didx.at[slot, row0]], s0).wait()
            pltpu.make_async_copy(r1, aggsh.at[didx.at[slot, row0 + 1]],
                                  s1).wait()

            @pl.when(jnp.logical_and(jj % 8 == 7, jj < 8 * (NBLK - 2)))
            def _():
                _load_block(jj // 8 + 2, slot, isem)

            @pl.when(jj % 8 == 7)
            def _():
                _wait_block(1 - slot, isem)

            nslot = ((jj + 1) // 8) % 2
            nrow0 = 2 * ((jj + 1) % 8)
            pltpu.async_copy(x_hbm.at[sidx.at[nslot, nrow0]], r0, g0)
            pltpu.async_copy(x_hbm.at[sidx.at[nslot, nrow0 + 1]], r1, g1)

        return carry

    lax.fori_loop(0, NITER, _steady, 0)
    pltpu.make_async_copy(r0, aggsh.at[didx.at[1, NI - 2]], s0).wait()
    pltpu.make_async_copy(r1, aggsh.at[didx.at[1, NI - 1]], s1).wait()
    plsc.subcore_barrier()

    outbase = c * N_PAD + rowbase
    pltpu.sync_copy(aggsh.at[pl.ds(rowbase, ROWS_PER_TILE)],
                    agg_hbm.at[pl.ds(outbase, ROWS_PER_TILE)])


BLK = 80
UOFF = N_PAD // BLK


def _dense_body(agg, x_u, x_i, wl_u, b_u, wr_u, wl_i, b_i, wr_i,
                out_u, out_i):
    f32 = jnp.float32
    out_u[...] = (jnp.dot(agg[1], wl_u[...], preferred_element_type=f32)
                  + jnp.dot(x_u[...], wr_u[...], preferred_element_type=f32)
                  + b_u[...])
    out_i[...] = (jnp.dot(agg[0], wl_i[...], preferred_element_type=f32)
                  + jnp.dot(x_i[...], wr_i[...], preferred_element_type=f32)
                  + b_i[...])


def _dense(agg_flat, x_user, x_item,
           WlT_u, b_u, WrT_u, WlT_i, b_i, WrT_i):
    agg3 = agg_flat.reshape(2, N_PAD, D)
    agg_spec = pl.BlockSpec((2, BLK, D), lambda i: (0, i, 0))
    row_spec = pl.BlockSpec((BLK, D), lambda i: (i, 0))
    w_spec = pl.BlockSpec((D, D), lambda i: (0, 0))
    b_spec = pl.BlockSpec((1, D), lambda i: (0, 0))
    return pl.pallas_call(
        _dense_body,
        grid=(N_NODE // BLK,),
        in_specs=[agg_spec, row_spec, row_spec,
                  w_spec, b_spec, w_spec, w_spec, b_spec, w_spec],
        out_specs=[row_spec, row_spec],
        out_shape=[jax.ShapeDtypeStruct((N_NODE, D), jnp.float32),
                   jax.ShapeDtypeStruct((N_NODE, D), jnp.float32)],
    )(agg3, x_user, x_item,
      WlT_u, b_u, WrT_u, WlT_i, b_i, WrT_i)


def kernel(x_user, x_item, edge_index_user_to_item, edge_index_item_to_user,
           W_l_u2i, b_l_u2i, W_r_u2i, W_l_i2u, b_l_i2u, W_r_i2u):
    x_all = jnp.concatenate([x_user, x_item], axis=0)
    src_all = jnp.concatenate([edge_index_user_to_item[0],
                               edge_index_item_to_user[0] + N_NODE]
                              ).reshape(2 * ROWS_PER_REL, CHUNK)
    dst_all = jnp.concatenate([edge_index_user_to_item[1],
                               edge_index_item_to_user[1]]
                              ).reshape(2 * ROWS_PER_REL, CHUNK)
    zeros_rows = jnp.zeros((ROWS_PER_TILE, D), jnp.float32)

    agg_flat = _sc_agg(x_all, src_all, dst_all, zeros_rows)

    out_user, out_item = _dense(
        agg_flat, x_user, x_item,
        W_l_i2u.T, b_l_i2u.reshape(1, D), W_r_i2u.T,
        W_l_u2i.T, b_l_u2i.reshape(1, D), W_r_u2i.T)
    return out_user, out_item


# speedup vs baseline: 1.1834x; 1.1825x over previous
import functools

import jax
import jax.numpy as jnp
from jax import lax
from jax.experimental import pallas as pl
from jax.experimental.pallas import tpu as pltpu
from jax.experimental.pallas import tpu_sc as plsc

N_NODE = 10000
D = 128
E_EDGES = 320000

NS = 16
SROWS = 624
LAST_BASE = SROWS * (NS - 1)
LAST_ROWS = N_NODE - LAST_BASE
CHUNK = 125
ROWS_PER_REL = E_EDGES // CHUNK
CHUNKS_PER_TILE = ROWS_PER_REL // NS
NI = 16
NBLK = CHUNKS_PER_TILE // NI
NITER = CHUNKS_PER_TILE // 2


@functools.partial(
    pl.kernel,
    mesh=plsc.VectorSubcoreMesh(core_axis_name="c", subcore_axis_name="s"),
    out_type=jax.ShapeDtypeStruct((2 * N_NODE, D), jnp.float32),
    scratch_types=[
        pltpu.VMEM((2, NI, CHUNK), jnp.int32),
        pltpu.VMEM((2, NI, CHUNK), jnp.int32),
        pltpu.VMEM((CHUNK, D), jnp.float32),
        pltpu.VMEM((CHUNK, D), jnp.float32),
        pltpu.VMEM_SHARED((N_NODE, D), jnp.float32),
        pltpu.SemaphoreType.DMA,
        pltpu.SemaphoreType.DMA,
        pltpu.SemaphoreType.DMA,
        pltpu.SemaphoreType.DMA,
        pltpu.SemaphoreType.DMA,
    ],
)
def _sc_agg(h_hbm, src_hbm, dst_hbm, base_hbm,
            out_hbm, sidx, didx, r0, r1, aggsh, g0, g1, s0, s1, isem):
    c = lax.axis_index("c")
    s = lax.axis_index("s")

    rowbase = s * SROWS

    @pl.when(s < NS - 1)
    def _():
        pltpu.sync_copy(base_hbm.at[pl.ds(c * N_NODE + s * SROWS, SROWS)],
                        aggsh.at[pl.ds(s * SROWS, SROWS)])

    @pl.when(s == NS - 1)
    def _():
        pltpu.sync_copy(base_hbm.at[pl.ds(c * N_NODE + LAST_BASE, LAST_ROWS)],
                        aggsh.at[pl.ds(LAST_BASE, LAST_ROWS)])

    plsc.subcore_barrier()

    tbase = c * ROWS_PER_REL + s * CHUNKS_PER_TILE

    def _load_block(b, slot, sem):
        pltpu.async_copy(src_hbm.at[pl.ds(tbase + NI * b, NI)], sidx.at[slot],
                         sem)
        pltpu.async_copy(dst_hbm.at[pl.ds(tbase + NI * b, NI)], didx.at[slot],
                         sem)

    def _wait_block(slot, sem):
        pltpu.make_async_copy(src_hbm.at[pl.ds(tbase, NI)], sidx.at[slot],
                              sem).wait()
        pltpu.make_async_copy(dst_hbm.at[pl.ds(tbase, NI)], didx.at[slot],
                              sem).wait()

    _load_block(0, 0, isem)
    _wait_block(0, isem)
    _load_block(1, 1, isem)

    pltpu.async_copy(h_hbm.at[sidx.at[0, 0]], r0, g0)
    pltpu.async_copy(h_hbm.at[sidx.at[0, 1]], r1, g1)

    def _steady(jj, carry):
        slot = (jj // 8) % 2
        row0 = 2 * (jj % 8)
        pltpu.make_async_copy(h_hbm.at[sidx.at[slot, row0]], r0, g0).wait()
        pltpu.async_copy(r0, aggsh.at[didx.at[slot, row0]], s0, add=True)
        pltpu.make_async_copy(h_hbm.at[sidx.at[slot, row0 + 1]], r1, g1).wait()
        pltpu.async_copy(r1, aggsh.at[didx.at[slot, row0 + 1]], s1, add=True)

        @pl.when(jj < NITER - 1)
        def _():
            pltpu.make_async_copy(r0, aggsh.at[didx.at[slot, row0]], s0).wait()
            pltpu.make_async_copy(r1, aggsh.at[didx.at[slot, row0 + 1]],
                                  s1).wait()

            @pl.when(jnp.logical_and(jj % 8 == 7, jj < 8 * (NBLK - 2)))
            def _():
                _load_block(jj // 8 + 2, slot, isem)

            @pl.when(jj % 8 == 7)
            def _():
                _wait_block(1 - slot, isem)

            nslot = ((jj + 1) // 8) % 2
            nrow0 = 2 * ((jj + 1) % 8)
            pltpu.async_copy(h_hbm.at[sidx.at[nslot, nrow0]], r0, g0)
            pltpu.async_copy(h_hbm.at[sidx.at[nslot, nrow0 + 1]], r1, g1)

        return carry

    lax.fori_loop(0, NITER, _steady, 0)
    pltpu.make_async_copy(r0, aggsh.at[didx.at[1, NI - 2]], s0).wait()
    pltpu.make_async_copy(r1, aggsh.at[didx.at[1, NI - 1]], s1).wait()
    plsc.subcore_barrier()

    @pl.when(s < NS - 1)
    def _():
        pltpu.sync_copy(aggsh.at[pl.ds(s * SROWS, SROWS)],
                        out_hbm.at[pl.ds(c * N_NODE + s * SROWS, SROWS)])

    @pl.when(s == NS - 1)
    def _():
        pltpu.sync_copy(aggsh.at[pl.ds(LAST_BASE, LAST_ROWS)],
                        out_hbm.at[pl.ds(c * N_NODE + LAST_BASE, LAST_ROWS)])


BLK = 1000


def _dense_body(x_u, x_i, wl_u2i, wr_u2i, b_u2i, wl_i2u, wr_i2u, b_i2u,
                h, base):
    f32 = jnp.float32
    h[0] = jnp.dot(x_u[...], wl_u2i[...], preferred_element_type=f32)
    h[1] = jnp.dot(x_i[...], wl_i2u[...], preferred_element_type=f32)
    base[0] = (jnp.dot(x_i[...], wr_u2i[...], preferred_element_type=f32)
               + b_u2i[...])
    base[1] = (jnp.dot(x_u[...], wr_i2u[...], preferred_element_type=f32)
               + b_i2u[...])


def _dense_pre(x_user, x_item, WlT_u2i, WrT_u2i, b_u2i, WlT_i2u, WrT_i2u,
               b_i2u):
    row_spec = pl.BlockSpec((BLK, D), lambda i: (i, 0))
    pair_spec = pl.BlockSpec((2, BLK, D), lambda i: (0, i, 0))
    w_spec = pl.BlockSpec((D, D), lambda i: (0, 0))
    b_spec = pl.BlockSpec((1, D), lambda i: (0, 0))
    return pl.pallas_call(
        _dense_body,
        grid=(N_NODE // BLK,),
        in_specs=[row_spec, row_spec,
                  w_spec, w_spec, b_spec, w_spec, w_spec, b_spec],
        out_specs=[pair_spec, pair_spec],
        out_shape=[jax.ShapeDtypeStruct((2, N_NODE, D), jnp.float32),
                   jax.ShapeDtypeStruct((2, N_NODE, D), jnp.float32)],
    )(x_user, x_item, WlT_u2i, WrT_u2i, b_u2i, WlT_i2u, WrT_i2u, b_i2u)


def kernel(x_user, x_item, edge_index_user_to_item, edge_index_item_to_user,
           W_l_u2i, b_l_u2i, W_r_u2i, W_l_i2u, b_l_i2u, W_r_i2u):
    h, base = _dense_pre(
        x_user, x_item,
        W_l_u2i.T, W_r_u2i.T, b_l_u2i.reshape(1, D),
        W_l_i2u.T, W_r_i2u.T, b_l_i2u.reshape(1, D))
    h_flat = h.reshape(2 * N_NODE, D)
    base_flat = base.reshape(2 * N_NODE, D)

    src_all = jnp.concatenate([edge_index_user_to_item[0],
                               edge_index_item_to_user[0] + N_NODE]
                              ).reshape(2 * ROWS_PER_REL, CHUNK)
    dst_all = jnp.concatenate([edge_index_user_to_item[1],
                               edge_index_item_to_user[1]]
                              ).reshape(2 * ROWS_PER_REL, CHUNK)

    out_flat = _sc_agg(h_flat, src_all, dst_all, base_flat)
    return out_flat[N_NODE:], out_flat[:N_NODE]


# speedup vs baseline: 1.2020x; 1.0157x over previous
import functools

import jax
import jax.numpy as jnp
from jax import lax
from jax.experimental import pallas as pl
from jax.experimental.pallas import tpu as pltpu
from jax.experimental.pallas import tpu_sc as plsc

N_NODE = 10000
D = 128
E_EDGES = 320000

NS = 16
SROWS = 624
LAST_BASE = SROWS * (NS - 1)
LAST_ROWS = N_NODE - LAST_BASE
CHUNK = 125
ROWS_PER_REL = E_EDGES // CHUNK
CHUNKS_PER_TILE = ROWS_PER_REL // NS
NI = 16
NBLK = CHUNKS_PER_TILE // NI
NITER = CHUNKS_PER_TILE // 2


@functools.partial(
    pl.kernel,
    mesh=plsc.VectorSubcoreMesh(core_axis_name="c", subcore_axis_name="s"),
    out_type=jax.ShapeDtypeStruct((2 * N_NODE, D), jnp.float32),
    scratch_types=[
        pltpu.VMEM((2, NI, CHUNK), jnp.int32),
        pltpu.VMEM((2, NI, CHUNK), jnp.int32),
        pltpu.VMEM((CHUNK, D), jnp.float32),
        pltpu.VMEM((CHUNK, D), jnp.float32),
        pltpu.VMEM_SHARED((N_NODE, D), jnp.float32),
        pltpu.SemaphoreType.DMA,
        pltpu.SemaphoreType.DMA,
        pltpu.SemaphoreType.DMA,
        pltpu.SemaphoreType.DMA,
        pltpu.SemaphoreType.DMA,
    ],
)
def _sc_agg(h_hbm, src_hbm, dst_hbm, base_hbm,
            out_hbm, sidx, didx, r0, r1, aggsh, g0, g1, s0, s1, isem):
    c = lax.axis_index("c")
    s = lax.axis_index("s")

    rowbase = s * SROWS

    @pl.when(s < NS - 1)
    def _():
        pltpu.sync_copy(base_hbm.at[pl.ds(c * N_NODE + s * SROWS, SROWS)],
                        aggsh.at[pl.ds(s * SROWS, SROWS)])

    @pl.when(s == NS - 1)
    def _():
        pltpu.sync_copy(base_hbm.at[pl.ds(c * N_NODE + LAST_BASE, LAST_ROWS)],
                        aggsh.at[pl.ds(LAST_BASE, LAST_ROWS)])

    plsc.subcore_barrier()

    tbase = c * ROWS_PER_REL + s * CHUNKS_PER_TILE

    def _load_block(b, slot, sem):
        pltpu.async_copy(src_hbm.at[pl.ds(tbase + NI * b, NI)], sidx.at[slot],
                         sem)
        pltpu.async_copy(dst_hbm.at[pl.ds(tbase + NI * b, NI)], didx.at[slot],
                         sem)

    def _wait_block(slot, sem):
        pltpu.make_async_copy(src_hbm.at[pl.ds(tbase, NI)], sidx.at[slot],
                              sem).wait()
        pltpu.make_async_copy(dst_hbm.at[pl.ds(tbase, NI)], didx.at[slot],
                              sem).wait()

    _load_block(0, 0, isem)
    _wait_block(0, isem)
    _load_block(1, 1, isem)

    pltpu.async_copy(h_hbm.at[sidx.at[0, 0]], r0, g0)
    pltpu.async_copy(h_hbm.at[sidx.at[0, 1]], r1, g1)

    def _steady(jj, carry):
        slot = (jj // 8) % 2
        row0 = 2 * (jj % 8)
        pltpu.make_async_copy(h_hbm.at[sidx.at[slot, row0]], r0, g0).wait()
        pltpu.async_copy(r0, aggsh.at[didx.at[slot, row0]], s0, add=True)
        pltpu.make_async_copy(h_hbm.at[sidx.at[slot, row0 + 1]], r1, g1).wait()
        pltpu.async_copy(r1, aggsh.at[didx.at[slot, row0 + 1]], s1, add=True)

        @pl.when(jj < NITER - 1)
        def _():
            @pl.when(jj % 8 == 7)
            def _():
                _wait_block(1 - slot, isem)

            nslot = ((jj + 1) // 8) % 2
            nrow0 = 2 * ((jj + 1) % 8)
            pltpu.make_async_copy(r0, aggsh.at[didx.at[slot, row0]], s0).wait()
            pltpu.async_copy(h_hbm.at[sidx.at[nslot, nrow0]], r0, g0)
            pltpu.make_async_copy(r1, aggsh.at[didx.at[slot, row0 + 1]],
                                  s1).wait()
            pltpu.async_copy(h_hbm.at[sidx.at[nslot, nrow0 + 1]], r1, g1)

            @pl.when(jnp.logical_and(jj % 8 == 7, jj < 8 * (NBLK - 2)))
            def _():
                _load_block(jj // 8 + 2, slot, isem)

        return carry

    lax.fori_loop(0, NITER, _steady, 0)
    pltpu.make_async_copy(r0, aggsh.at[didx.at[1, NI - 2]], s0).wait()
    pltpu.make_async_copy(r1, aggsh.at[didx.at[1, NI - 1]], s1).wait()
    plsc.subcore_barrier()

    @pl.when(s < NS - 1)
    def _():
        pltpu.sync_copy(aggsh.at[pl.ds(s * SROWS, SROWS)],
                        out_hbm.at[pl.ds(c * N_NODE + s * SROWS, SROWS)])

    @pl.when(s == NS - 1)
    def _():
        pltpu.sync_copy(aggsh.at[pl.ds(LAST_BASE, LAST_ROWS)],
                        out_hbm.at[pl.ds(c * N_NODE + LAST_BASE, LAST_ROWS)])


BLK = 1000


def _dense_body(x_u, x_i, wl_u2i, wr_u2i, b_u2i, wl_i2u, wr_i2u, b_i2u,
                h, base):
    f32 = jnp.float32
    h[0] = jnp.dot(x_u[...], wl_u2i[...], preferred_element_type=f32)
    h[1] = jnp.dot(x_i[...], wl_i2u[...], preferred_element_type=f32)
    base[0] = (jnp.dot(x_i[...], wr_u2i[...], preferred_element_type=f32)
               + b_u2i[...])
    base[1] = (jnp.dot(x_u[...], wr_i2u[...], preferred_element_type=f32)
               + b_i2u[...])


def _dense_pre(x_user, x_item, WlT_u2i, WrT_u2i, b_u2i, WlT_i2u, WrT_i2u,
               b_i2u):
    row_spec = pl.BlockSpec((BLK, D), lambda i: (i, 0))
    pair_spec = pl.BlockSpec((2, BLK, D), lambda i: (0, i, 0))
    w_spec = pl.BlockSpec((D, D), lambda i: (0, 0))
    b_spec = pl.BlockSpec((1, D), lambda i: (0, 0))
    return pl.pallas_call(
        _dense_body,
        grid=(N_NODE // BLK,),
        in_specs=[row_spec, row_spec,
                  w_spec, w_spec, b_spec, w_spec, w_spec, b_spec],
        out_specs=[pair_spec, pair_spec],
        out_shape=[jax.ShapeDtypeStruct((2, N_NODE, D), jnp.float32),
                   jax.ShapeDtypeStruct((2, N_NODE, D), jnp.float32)],
    )(x_user, x_item, WlT_u2i, WrT_u2i, b_u2i, WlT_i2u, WrT_i2u, b_i2u)


def kernel(x_user, x_item, edge_index_user_to_item, edge_index_item_to_user,
           W_l_u2i, b_l_u2i, W_r_u2i, W_l_i2u, b_l_i2u, W_r_i2u):
    h, base = _dense_pre(
        x_user, x_item,
        W_l_u2i.T, W_r_u2i.T, b_l_u2i.reshape(1, D),
        W_l_i2u.T, W_r_i2u.T, b_l_i2u.reshape(1, D))
    h_flat = h.reshape(2 * N_NODE, D)
    base_flat = base.reshape(2 * N_NODE, D)

    src_all = jnp.concatenate([edge_index_user_to_item[0],
                               edge_index_item_to_user[0] + N_NODE]
                              ).reshape(2 * ROWS_PER_REL, CHUNK)
    dst_all = jnp.concatenate([edge_index_user_to_item[1],
                               edge_index_item_to_user[1]]
                              ).reshape(2 * ROWS_PER_REL, CHUNK)

    out_flat = _sc_agg(h_flat, src_all, dst_all, base_flat)
    return out_flat[N_NODE:], out_flat[:N_NODE]


# speedup vs baseline: 1.2111x; 1.0076x over previous
import functools

import jax
import jax.numpy as jnp
from jax import lax
from jax.experimental import pallas as pl
from jax.experimental.pallas import tpu as pltpu
from jax.experimental.pallas import tpu_sc as plsc

N_NODE = 10000
D = 128
E_EDGES = 320000

NS = 16
SROWS = 624
LAST_BASE = SROWS * (NS - 1)
LAST_ROWS = N_NODE - LAST_BASE
CHUNK = 125
ROWS_PER_REL = E_EDGES // CHUNK
CHUNKS_PER_TILE = ROWS_PER_REL // NS
NI = 16
NBLK = CHUNKS_PER_TILE // NI
NITER = CHUNKS_PER_TILE // 2


@functools.partial(
    pl.kernel,
    mesh=plsc.VectorSubcoreMesh(core_axis_name="c", subcore_axis_name="s"),
    out_type=jax.ShapeDtypeStruct((2 * N_NODE, D), jnp.float32),
    scratch_types=[
        pltpu.VMEM((2, NI, CHUNK), jnp.int32),
        pltpu.VMEM((2, NI, CHUNK), jnp.int32),
        pltpu.VMEM((CHUNK, D), jnp.float32),
        pltpu.VMEM((CHUNK, D), jnp.float32),
        pltpu.VMEM_SHARED((N_NODE, D), jnp.float32),
        pltpu.SemaphoreType.DMA,
        pltpu.SemaphoreType.DMA,
        pltpu.SemaphoreType.DMA,
        pltpu.SemaphoreType.DMA,
        pltpu.SemaphoreType.DMA,
    ],
)
def _sc_agg(h_hbm, src_hbm, dst_hbm, base_hbm,
            out_hbm, sidx, didx, r0, r1, aggsh, g0, g1, s0, s1, isem):
    c = lax.axis_index("c")
    s = lax.axis_index("s")

    rowbase = s * SROWS

    @pl.when(s < NS - 1)
    def _():
        pltpu.sync_copy(base_hbm.at[pl.ds(c * N_NODE + s * SROWS, SROWS)],
                        aggsh.at[pl.ds(s * SROWS, SROWS)])

    @pl.when(s == NS - 1)
    def _():
        pltpu.sync_copy(base_hbm.at[pl.ds(c * N_NODE + LAST_BASE, LAST_ROWS)],
                        aggsh.at[pl.ds(LAST_BASE, LAST_ROWS)])

    plsc.subcore_barrier()

    tbase = c * ROWS_PER_REL + s * CHUNKS_PER_TILE

    def _load_block(b, slot, sem):
        pltpu.async_copy(src_hbm.at[pl.ds(tbase + NI * b, NI)], sidx.at[slot],
                         sem)
        pltpu.async_copy(dst_hbm.at[pl.ds(tbase + NI * b, NI)], didx.at[slot],
                         sem)

    def _wait_block(slot, sem):
        pltpu.make_async_copy(src_hbm.at[pl.ds(tbase, NI)], sidx.at[slot],
                              sem).wait()
        pltpu.make_async_copy(dst_hbm.at[pl.ds(tbase, NI)], didx.at[slot],
                              sem).wait()

    _load_block(0, 0, isem)
    _wait_block(0, isem)
    _load_block(1, 1, isem)

    pltpu.async_copy(h_hbm.at[sidx.at[0, 0]], r0, g0)
    pltpu.async_copy(h_hbm.at[sidx.at[0, 1]], r1, g1)

    def _steady(jj, carry):
        slot = (jj // 8) % 2
        row0 = 2 * (jj % 8)
        pltpu.make_async_copy(h_hbm.at[sidx.at[slot, row0]], r0, g0).wait()
        pltpu.async_copy(r0, aggsh.at[didx.at[slot, row0]], s0, add=True)
        pltpu.make_async_copy(h_hbm.at[sidx.at[slot, row0 + 1]], r1, g1).wait()
        pltpu.async_copy(r1, aggsh.at[didx.at[slot, row0 + 1]], s1, add=True)

        @pl.when(jj < NITER - 1)
        def _():
            @pl.when(jj % 8 == 7)
            def _():
                _wait_block(1 - slot, isem)

            nslot = ((jj + 1) // 8) % 2
            nrow0 = 2 * ((jj + 1) % 8)
            pltpu.make_async_copy(r0, aggsh.at[didx.at[slot, row0]], s0).wait()
            pltpu.async_copy(h_hbm.at[sidx.at[nslot, nrow0]], r0, g0)
            pltpu.make_async_copy(r1, aggsh.at[didx.at[slot, row0 + 1]],
                                  s1).wait()
            pltpu.async_copy(h_hbm.at[sidx.at[nslot, nrow0 + 1]], r1, g1)

            @pl.when(jnp.logical_and(jj % 8 == 7, jj < 8 * (NBLK - 2)))
            def _():
                _load_block(jj // 8 + 2, slot, isem)

        return carry

    lax.fori_loop(0, NITER, _steady, 0)
    pltpu.make_async_copy(r0, aggsh.at[didx.at[1, NI - 2]], s0).wait()
    pltpu.make_async_copy(r1, aggsh.at[didx.at[1, NI - 1]], s1).wait()
    plsc.subcore_barrier()

    @pl.when(s < NS - 1)
    def _():
        pltpu.sync_copy(aggsh.at[pl.ds(s * SROWS, SROWS)],
                        out_hbm.at[pl.ds(c * N_NODE + s * SROWS, SROWS)])

    @pl.when(s == NS - 1)
    def _():
        pltpu.sync_copy(aggsh.at[pl.ds(LAST_BASE, LAST_ROWS)],
                        out_hbm.at[pl.ds(c * N_NODE + LAST_BASE, LAST_ROWS)])


BLK = 1000


IBLK = ROWS_PER_REL // (N_NODE // BLK)


def _dense_body(x_u, x_i, wl_u2i, wr_u2i, b_u2i, wl_i2u, wr_i2u, b_i2u,
                s0_r, d0_r, s1_r, d1_r, h, base, sidx, didx):
    f32 = jnp.float32
    h[0] = jnp.dot(x_u[...], wl_u2i[...], preferred_element_type=f32)
    h[1] = jnp.dot(x_i[...], wl_i2u[...], preferred_element_type=f32)
    base[0] = (jnp.dot(x_i[...], wr_u2i[...], preferred_element_type=f32)
               + b_u2i[...])
    base[1] = (jnp.dot(x_u[...], wr_i2u[...], preferred_element_type=f32)
               + b_i2u[...])
    sidx[0] = s0_r[...]
    sidx[1] = s1_r[...] + N_NODE
    didx[0] = d0_r[...]
    didx[1] = d1_r[...]


def _dense_pre(x_user, x_item, WlT_u2i, WrT_u2i, b_u2i, WlT_i2u, WrT_i2u,
               b_i2u, s0, d0, s1, d1):
    row_spec = pl.BlockSpec((BLK, D), lambda i: (i, 0))
    pair_spec = pl.BlockSpec((2, BLK, D), lambda i: (0, i, 0))
    w_spec = pl.BlockSpec((D, D), lambda i: (0, 0))
    b_spec = pl.BlockSpec((1, D), lambda i: (0, 0))
    i_spec = pl.BlockSpec((IBLK, CHUNK), lambda i: (i, 0))
    ipair_spec = pl.BlockSpec((2, IBLK, CHUNK), lambda i: (0, i, 0))
    return pl.pallas_call(
        _dense_body,
        grid=(N_NODE // BLK,),
        in_specs=[row_spec, row_spec,
                  w_spec, w_spec, b_spec, w_spec, w_spec, b_spec,
                  i_spec, i_spec, i_spec, i_spec],
        out_specs=[pair_spec, pair_spec, ipair_spec, ipair_spec],
        out_shape=[jax.ShapeDtypeStruct((2, N_NODE, D), jnp.float32),
                   jax.ShapeDtypeStruct((2, N_NODE, D), jnp.float32),
                   jax.ShapeDtypeStruct((2, ROWS_PER_REL, CHUNK), jnp.int32),
                   jax.ShapeDtypeStruct((2, ROWS_PER_REL, CHUNK), jnp.int32)],
    )(x_user, x_item, WlT_u2i, WrT_u2i, b_u2i, WlT_i2u, WrT_i2u, b_i2u,
      s0, d0, s1, d1)


def kernel(x_user, x_item, edge_index_user_to_item, edge_index_item_to_user,
           W_l_u2i, b_l_u2i, W_r_u2i, W_l_i2u, b_l_i2u, W_r_i2u):
    h, base, src_all, dst_all = _dense_pre(
        x_user, x_item,
        W_l_u2i.T, W_r_u2i.T, b_l_u2i.reshape(1, D),
        W_l_i2u.T, W_r_i2u.T, b_l_i2u.reshape(1, D),
        edge_index_user_to_item[0].reshape(ROWS_PER_REL, CHUNK),
        edge_index_user_to_item[1].reshape(ROWS_PER_REL, CHUNK),
        edge_index_item_to_user[0].reshape(ROWS_PER_REL, CHUNK),
        edge_index_item_to_user[1].reshape(ROWS_PER_REL, CHUNK))
    h_flat = h.reshape(2 * N_NODE, D)
    base_flat = base.reshape(2 * N_NODE, D)

    out_flat = _sc_agg(h_flat,
                       src_all.reshape(2 * ROWS_PER_REL, CHUNK),
                       dst_all.reshape(2 * ROWS_PER_REL, CHUNK),
                       base_flat)
    return out_flat[N_NODE:], out_flat[:N_NODE]
